# baseline (speedup 1.0000x reference)
import functools

import jax
import jax.numpy as jnp
from jax import lax
from jax.experimental import pallas as pl
from jax.experimental.pallas import tpu as pltpu
from jax.experimental.pallas import tpu_sc as plsc

B = 4096
L = 50
D = 128
N = B * L
TYPE_VOCAB = 9
REF_VOCAB = 51

_info = plsc.get_sparse_core_info()
NC = _info.num_cores
NS = _info.num_subcores
NW = NC * NS
TOK_PER_W = N // NW
C = 400
NCHUNK = TOK_PER_W // C
HALF = NCHUNK // 2
G = C // 16

_mesh = plsc.VectorSubcoreMesh(core_axis_name="c", subcore_axis_name="s")


@functools.partial(
    pl.kernel,
    mesh=_mesh,
    out_type=jax.ShapeDtypeStruct((N, D), jnp.float32),
    scratch_types=[
        pltpu.VMEM((C,), jnp.int32),
        pltpu.VMEM((C,), jnp.int32),
        pltpu.VMEM((C,), jnp.int32),
        pltpu.VMEM((C,), jnp.int32),
        pltpu.VMEM((C,), jnp.int32),
        pltpu.VMEM((C,), jnp.int32),
        pltpu.VMEM((C, D), jnp.float32),
        pltpu.VMEM((C, D), jnp.float32),
        pltpu.VMEM((TYPE_VOCAB, D), jnp.float32),
        pltpu.VMEM((REF_VOCAB, D), jnp.float32),
        pltpu.SemaphoreType.DMA,
        pltpu.SemaphoreType.DMA,
        pltpu.SemaphoreType.DMA,
        pltpu.SemaphoreType.DMA,
        pltpu.SemaphoreType.DMA,
        pltpu.SemaphoreType.DMA,
        pltpu.SemaphoreType.DMA,
        pltpu.SemaphoreType.DMA,
    ],
)
def _cad_embed(tids, pids, rids, ttab, ptab, rtab, out,
               pidx0, pidx1, tidx0, tidx1, ridx0, ridx1, prow0, prow1,
               ttab_v, rtab_v, sg0, sg1, so0, so1, sit0, sit1, sir0, sir1):
    wid = lax.axis_index("s") * NC + lax.axis_index("c")
    base = wid * TOK_PER_W
    pidx = (pidx0, pidx1)
    tidx = (tidx0, tidx1)
    ridx = (ridx0, ridx1)
    prow = (prow0, prow1)
    sg = (sg0, sg1)
    so = (so0, so1)
    sit = (sit0, sit1)
    sir = (sir0, sir1)

    pltpu.sync_copy(ttab, ttab_v)
    pltpu.sync_copy(rtab, rtab_v)

    pltpu.sync_copy(pids.at[pl.ds(base, C)], pidx0)
    pltpu.async_copy(ptab.at[pidx0], prow0, sg0)
    pltpu.async_copy(tids.at[pl.ds(base, C)], tidx0, sit0)
    pltpu.async_copy(rids.at[pl.ds(base, C)], ridx0, sir0)

    def iter_body(i, carry):
        for b in range(2):
            k = 2 * i + b
            off = base + k * C
            nb = 1 - b

            if b == 0:
                @pl.when(i >= 1)
                def _():
                    pltpu.make_async_copy(
                        prow[nb], out.at[pl.ds(base, C)], so[nb]).wait()

                pltpu.sync_copy(pids.at[pl.ds(off + C, C)], pidx[nb])
                pltpu.async_copy(ptab.at[pidx[nb]], prow[nb], sg[nb])
                pltpu.async_copy(tids.at[pl.ds(off + C, C)], tidx[nb], sit[nb])
                pltpu.async_copy(rids.at[pl.ds(off + C, C)], ridx[nb], sir[nb])
            else:
                @pl.when(i < HALF - 1)
                def _():
                    pltpu.make_async_copy(
                        prow[nb], out.at[pl.ds(base, C)], so[nb]).wait()
                    pltpu.sync_copy(pids.at[pl.ds(off + C, C)], pidx[nb])
                    pltpu.async_copy(ptab.at[pidx[nb]], prow[nb], sg[nb])
                    pltpu.async_copy(
                        tids.at[pl.ds(off + C, C)], tidx[nb], sit[nb])
                    pltpu.async_copy(
                        rids.at[pl.ds(off + C, C)], ridx[nb], sir[nb])

            pltpu.make_async_copy(ptab.at[pl.ds(0, C)], prow[b], sg[b]).wait()
            pltpu.make_async_copy(tids.at[pl.ds(off, C)], tidx[b], sit[b]).wait()
            pltpu.make_async_copy(rids.at[pl.ds(off, C)], ridx[b], sir[b]).wait()
            tidx_v = tidx[b]
            ridx_v = ridx[b]
            prow_b = prow[b]

            @plsc.parallel_loop(0, G)
            def _(g):
                tv = tidx_v[pl.ds(g * 16, 16)]
                rv = ridx_v[pl.ds(g * 16, 16)]
                for j in range(16):
                    row = g * 16 + j
                    ts = tv[j]
                    rs = rv[j]
                    for cb in range(D // 16):
                        sl = pl.ds(cb * 16, 16)
                        plsc.addupdate(prow_b.at[row, sl],
                                       ttab_v[ts, sl] + rtab_v[rs, sl])

            pltpu.async_copy(prow_b, out.at[pl.ds(off, C)], so[b])
        return carry

    lax.fori_loop(0, HALF, iter_body, 0)

    pltpu.make_async_copy(prow0, out.at[pl.ds(base, C)], so0).wait()
    pltpu.make_async_copy(prow1, out.at[pl.ds(base, C)], so1).wait()


def kernel(type_ids, posi_ids, ref_ids, type_table, posi_table, ref_table):
    out = _cad_embed(
        type_ids.reshape(N),
        posi_ids.reshape(N),
        ref_ids.reshape(N),
        type_table,
        posi_table,
        ref_table,
    )
    return out.reshape(B, L, D)


# speedup vs baseline: 1.0358x; 1.0358x over previous
import functools

import jax
import jax.numpy as jnp
from jax import lax
from jax.experimental import pallas as pl
from jax.experimental.pallas import tpu as pltpu
from jax.experimental.pallas import tpu_sc as plsc

B = 4096
L = 50
D = 128
N = B * L
TYPE_VOCAB = 9
REF_VOCAB = 51

_info = plsc.get_sparse_core_info()
NC = _info.num_cores
NS = _info.num_subcores
NW = NC * NS
TOK_PER_W = N // NW
C = 400
NCHUNK = TOK_PER_W // C
HALF = NCHUNK // 2
G = C // 16

_mesh = plsc.VectorSubcoreMesh(core_axis_name="c", subcore_axis_name="s")


@functools.partial(
    pl.kernel,
    mesh=_mesh,
    out_type=jax.ShapeDtypeStruct((N, D), jnp.float32),
    scratch_types=[
        pltpu.VMEM((TOK_PER_W,), jnp.int32),
        pltpu.VMEM((TOK_PER_W,), jnp.int32),
        pltpu.VMEM((TOK_PER_W,), jnp.int32),
        pltpu.VMEM((C, D), jnp.float32),
        pltpu.VMEM((C, D), jnp.float32),
        pltpu.VMEM((TYPE_VOCAB, D), jnp.float32),
        pltpu.VMEM((REF_VOCAB, D), jnp.float32),
        pltpu.SemaphoreType.DMA,
        pltpu.SemaphoreType.DMA,
        pltpu.SemaphoreType.DMA,
        pltpu.SemaphoreType.DMA,
    ],
)
def _cad_embed(tids, pids, rids, ttab, ptab, rtab, out,
               pidx_a, tidx_a, ridx_a, prow0, prow1,
               ttab_v, rtab_v, sg0, sg1, so0, so1):
    wid = lax.axis_index("s") * NC + lax.axis_index("c")
    base = wid * TOK_PER_W
    prow = (prow0, prow1)
    sg = (sg0, sg1)
    so = (so0, so1)

    pltpu.sync_copy(ttab, ttab_v)
    pltpu.sync_copy(rtab, rtab_v)

    pltpu.sync_copy(pids.at[pl.ds(base, TOK_PER_W)], pidx_a)
    pltpu.async_copy(ptab.at[pidx_a.at[pl.ds(0, C)]], prow0, sg0)
    pltpu.sync_copy(tids.at[pl.ds(base, TOK_PER_W)], tidx_a)
    pltpu.sync_copy(rids.at[pl.ds(base, TOK_PER_W)], ridx_a)

    def iter_body(i, carry):
        for b in range(2):
            k = 2 * i + b
            off = k * C
            nb = 1 - b

            if b == 0:
                @pl.when(i >= 1)
                def _():
                    pltpu.make_async_copy(
                        prow[nb], out.at[pl.ds(base, C)], so[nb]).wait()

                pltpu.async_copy(
                    ptab.at[pidx_a.at[pl.ds(off + C, C)]], prow[nb], sg[nb])
            else:
                @pl.when(i < HALF - 1)
                def _():
                    pltpu.make_async_copy(
                        prow[nb], out.at[pl.ds(base, C)], so[nb]).wait()
                    pltpu.async_copy(
                        ptab.at[pidx_a.at[pl.ds(off + C, C)]],
                        prow[nb], sg[nb])

            pltpu.make_async_copy(ptab.at[pl.ds(0, C)], prow[b], sg[b]).wait()
            prow_b = prow[b]

            @plsc.parallel_loop(0, G)
            def _(g):
                tv = tidx_a[pl.ds(off + g * 16, 16)]
                rv = ridx_a[pl.ds(off + g * 16, 16)]
                for j in range(16):
                    row = g * 16 + j
                    ts = tv[j]
                    rs = rv[j]
                    for cb in range(D // 16):
                        sl = pl.ds(cb * 16, 16)
                        plsc.addupdate(prow_b.at[row, sl],
                                       ttab_v[ts, sl] + rtab_v[rs, sl])

            pltpu.async_copy(prow_b, out.at[pl.ds(base + off, C)], so[b])
        return carry

    lax.fori_loop(0, HALF, iter_body, 0)

    pltpu.make_async_copy(prow0, out.at[pl.ds(base, C)], so0).wait()
    pltpu.make_async_copy(prow1, out.at[pl.ds(base, C)], so1).wait()


def kernel(type_ids, posi_ids, ref_ids, type_table, posi_table, ref_table):
    out = _cad_embed(
        type_ids.reshape(N),
        posi_ids.reshape(N),
        ref_ids.reshape(N),
        type_table,
        posi_table,
        ref_table,
    )
    return out.reshape(B, L, D)


# speedup vs baseline: 1.5771x; 1.5227x over previous
import functools

import jax
import jax.numpy as jnp
from jax import lax
from jax.experimental import pallas as pl
from jax.experimental.pallas import tpu as pltpu
from jax.experimental.pallas import tpu_sc as plsc

B = 4096
L = 50
D = 128
N = B * L
TYPE_VOCAB = 9
REF_VOCAB = 51

_info = plsc.get_sparse_core_info()
NC = _info.num_cores
NS = _info.num_subcores
NW = NC * NS
TOK_PER_W = N // NW
C = 400
NCHUNK = TOK_PER_W // C
HALF = NCHUNK // 2
G = C // 16
ROWS_PER_CHUNK = C // L
ROWS_PER_W = TOK_PER_W // L

_mesh = plsc.VectorSubcoreMesh(core_axis_name="c", subcore_axis_name="s")


@functools.partial(
    pl.kernel,
    mesh=_mesh,
    out_type=jax.ShapeDtypeStruct((B, L, D), jnp.float32),
    scratch_types=[
        pltpu.VMEM((TOK_PER_W,), jnp.int32),
        pltpu.VMEM((TOK_PER_W,), jnp.int32),
        pltpu.VMEM((TOK_PER_W,), jnp.int32),
        pltpu.VMEM((C, D), jnp.float32),
        pltpu.VMEM((C, D), jnp.float32),
        pltpu.VMEM((TYPE_VOCAB, D), jnp.float32),
        pltpu.VMEM((REF_VOCAB, D), jnp.float32),
        pltpu.SemaphoreType.DMA,
        pltpu.SemaphoreType.DMA,
        pltpu.SemaphoreType.DMA,
        pltpu.SemaphoreType.DMA,
    ],
)
def _cad_embed(tids, pids, rids, ttab, ptab, rtab, out,
               pidx_a, tidx_a, ridx_a, prow0, prow1,
               ttab_v, rtab_v, sg0, sg1, so0, so1):
    wid = lax.axis_index("s") * NC + lax.axis_index("c")
    base = wid * TOK_PER_W
    brow_base = wid * ROWS_PER_W
    prow = (prow0, prow1)
    sg = (sg0, sg1)
    so = (so0, so1)

    pltpu.sync_copy(ttab, ttab_v)
    pltpu.sync_copy(rtab, rtab_v)

    def drain_out(buf, sem):
        for r in range(ROWS_PER_CHUNK):
            pltpu.make_async_copy(
                buf.at[pl.ds(r * L, L)], out.at[brow_base], sem).wait()

    pltpu.sync_copy(pids.at[pl.ds(base, TOK_PER_W)], pidx_a)
    pltpu.async_copy(ptab.at[pidx_a.at[pl.ds(0, C)]], prow0, sg0)
    pltpu.sync_copy(tids.at[pl.ds(base, TOK_PER_W)], tidx_a)
    pltpu.sync_copy(rids.at[pl.ds(base, TOK_PER_W)], ridx_a)

    def iter_body(i, carry):
        for b in range(2):
            k = 2 * i + b
            off = k * C
            nb = 1 - b

            if b == 0:
                @pl.when(i >= 1)
                def _():
                    drain_out(prow[nb], so[nb])

                pltpu.async_copy(
                    ptab.at[pidx_a.at[pl.ds(off + C, C)]], prow[nb], sg[nb])
            else:
                @pl.when(i < HALF - 1)
                def _():
                    drain_out(prow[nb], so[nb])
                    pltpu.async_copy(
                        ptab.at[pidx_a.at[pl.ds(off + C, C)]],
                        prow[nb], sg[nb])

            pltpu.make_async_copy(ptab.at[pl.ds(0, C)], prow[b], sg[b]).wait()
            prow_b = prow[b]

            @plsc.parallel_loop(0, G)
            def _(g):
                tv = tidx_a[pl.ds(off + g * 16, 16)]
                rv = ridx_a[pl.ds(off + g * 16, 16)]
                for j in range(16):
                    row = g * 16 + j
                    ts = tv[j]
                    rs = rv[j]
                    for cb in range(D // 16):
                        sl = pl.ds(cb * 16, 16)
                        plsc.addupdate(prow_b.at[row, sl],
                                       ttab_v[ts, sl] + rtab_v[rs, sl])

            brow0 = brow_base + k * ROWS_PER_CHUNK
            for r in range(ROWS_PER_CHUNK):
                pltpu.async_copy(
                    prow_b.at[pl.ds(r * L, L)], out.at[brow0 + r], so[b])
        return carry

    lax.fori_loop(0, HALF, iter_body, 0)

    drain_out(prow0, so0)
    drain_out(prow1, so1)


def kernel(type_ids, posi_ids, ref_ids, type_table, posi_table, ref_table):
    return _cad_embed(
        type_ids.reshape(N),
        posi_ids.reshape(N),
        ref_ids.reshape(N),
        type_table,
        posi_table,
        ref_table,
    )
